# baseline (speedup 1.0000x reference)
import functools

import jax
import jax.numpy as jnp
from jax import lax
from jax.experimental import pallas as pl
from jax.experimental.pallas import tpu as pltpu
from jax.experimental.pallas import tpu_sc as plsc

VOCAB_N = 100000
EMB_N = 32
CTX_N = 20
BATCH_N = 1024

_NH = 2
_BH = BATCH_N // _NH

_NC = 2
_NS = 16
_NW = _NC * _NS
_BPW = _BH // _NW
_RPW = _BPW * CTX_N

_VT = 2048
_NT = 49
_PAD_COLS = _VT * _NT - VOCAB_N


def _sc_body(idx_hbm, u_hbm, out_hbm, idx_v, rows_v, acc_v, sem):
    wid = lax.axis_index("s") * _NC + lax.axis_index("c")
    pltpu.sync_copy(idx_hbm.at[:, pl.ds(wid * _BPW, _BPW)], idx_v)
    cps = [
        pltpu.async_copy(
            u_hbm.at[idx_v.at[c]], rows_v.at[pl.ds(c * _BPW, _BPW)], sem
        )
        for c in range(CTX_N)
    ]
    for cp in cps:
        cp.wait()
    for j in range(_BPW):
        a0 = rows_v[j, 0:16]
        a1 = rows_v[j, 16:32]
        for c in range(1, CTX_N):
            a0 = a0 + rows_v[c * _BPW + j, 0:16]
            a1 = a1 + rows_v[c * _BPW + j, 16:32]
        acc_v[j, 0:16] = a0
        acc_v[j, 16:32] = a1
    pltpu.sync_copy(acc_v, out_hbm.at[pl.ds(wid * _BPW, _BPW)])


def _sc_embed_sum(idx2, u):
    return pl.kernel(
        _sc_body,
        out_type=jax.ShapeDtypeStruct((_BH, EMB_N), jnp.float32),
        mesh=plsc.VectorSubcoreMesh(core_axis_name="c", subcore_axis_name="s"),
        compiler_params=pltpu.CompilerParams(use_tc_tiling_on_sc=False),
        scratch_types=[
            pltpu.VMEM((CTX_N, _BPW), jnp.int32),
            pltpu.VMEM((_RPW, EMB_N), jnp.float32),
            pltpu.VMEM((_BPW, EMB_N), jnp.float32),
            pltpu.SemaphoreType.DMA,
        ],
    )(idx2, u)


def _lse_body(sums_ref, wt_ref, b_ref, lse_ref, s_ref):
    i = pl.program_id(0)

    @pl.when(i == 0)
    def _():
        s_ref[...] = jnp.zeros((1, _BH), jnp.float32)

    v0 = i * _VT
    valid_c = (lax.broadcasted_iota(jnp.int32, (1, _VT), 1) + v0) < VOCAB_N
    wt = jnp.where(valid_c, wt_ref[...], 0.0)
    b = jnp.where(valid_c, b_ref[...], 0.0).reshape(_VT, 1)
    x = lax.dot_general(
        wt, sums_ref[...], (((0,), (1,)), ((), ())),
        preferred_element_type=jnp.float32,
    ) + b
    s_ref[...] += jnp.sum(jnp.exp(x), axis=0, keepdims=True)

    @pl.when(i == _NT - 1)
    def _():
        lse_ref[...] = jnp.log(s_ref[...] - jnp.float32(_PAD_COLS))


def _lse_half(sums_h, wt, b2):
    return pl.pallas_call(
        _lse_body,
        grid=(_NT,),
        in_specs=[
            pl.BlockSpec((_BH, EMB_N), lambda i: (0, 0)),
            pl.BlockSpec((EMB_N, _VT), lambda i: (0, i)),
            pl.BlockSpec((1, _VT), lambda i: (0, i)),
        ],
        out_specs=pl.BlockSpec((1, _BH), lambda i: (0, 0)),
        out_shape=jax.ShapeDtypeStruct((1, _BH), jnp.float32),
        scratch_shapes=[
            pltpu.VMEM((1, _BH), jnp.float32),
        ],
    )(sums_h, wt, b2)


def _out_body(sums_ref, wt_ref, b_ref, lse_ref, o_ref):
    x = lax.dot_general(
        wt_ref[...], sums_ref[...], (((0,), (1,)), ((), ())),
        preferred_element_type=jnp.float32,
    )
    o_ref[...] = x + b_ref[...].reshape(_VT, 1) - lse_ref[...]


def kernel(inputs, U, W, b):
    idx2 = inputs.astype(jnp.int32)
    wt = W.T
    b2 = b.reshape(1, VOCAB_N)
    halves = []
    for h in range(_NH):
        idx_h = lax.slice_in_dim(idx2, h * _BH, (h + 1) * _BH, axis=1)
        sums_h = _sc_embed_sum(idx_h, U)
        lse_h = _lse_half(sums_h, wt, b2)
        halves.append((sums_h, lse_h))
    sums = jnp.concatenate([s for s, _ in halves], axis=0)
    lse = jnp.concatenate([l for _, l in halves], axis=1)
    out_t = pl.pallas_call(
        _out_body,
        grid=(_NT,),
        in_specs=[
            pl.BlockSpec((BATCH_N, EMB_N), lambda i: (0, 0)),
            pl.BlockSpec((EMB_N, _VT), lambda i: (0, i)),
            pl.BlockSpec((1, _VT), lambda i: (0, i)),
            pl.BlockSpec((1, BATCH_N), lambda i: (0, 0)),
        ],
        out_specs=pl.BlockSpec((_VT, BATCH_N), lambda i: (i, 0)),
        out_shape=jax.ShapeDtypeStruct((VOCAB_N, BATCH_N), jnp.float32),
    )(sums, wt, b2, lse)
    return out_t.T


# speedup vs baseline: 1.1267x; 1.1267x over previous
import jax
import jax.numpy as jnp
from jax import lax
from jax.experimental import pallas as pl
from jax.experimental.pallas import tpu as pltpu
from jax.experimental.pallas import tpu_sc as plsc

VOCAB_N = 100000
EMB_N = 32
CTX_N = 20
BATCH_N = 1024

_NC = 2
_NS = 16
_NW = _NC * _NS
_BPW = BATCH_N // _NW
_RPW = _BPW * CTX_N
_GCH = 128
_NG = _RPW // _GCH

_VT = 2048
_NT = 49
_PAD_COLS = _VT * _NT - VOCAB_N
_BT = 32
_NB = BATCH_N // _BT


def _sc_body(idx_hbm, u_hbm, out_hbm, idx_v, rows_v, acc_v, sem):
    wid = lax.axis_index("s") * _NC + lax.axis_index("c")
    pltpu.sync_copy(idx_hbm.at[:, pl.ds(wid * _BPW, _BPW)], idx_v)
    cps = [
        pltpu.async_copy(
            u_hbm.at[idx_v.at[c]], rows_v.at[pl.ds(c * _BPW, _BPW)], sem
        )
        for c in range(CTX_N)
    ]
    for cp in cps:
        cp.wait()
    for j in range(_BPW):
        a0 = rows_v[j, 0:16]
        a1 = rows_v[j, 16:32]
        for c in range(1, CTX_N):
            a0 = a0 + rows_v[c * _BPW + j, 0:16]
            a1 = a1 + rows_v[c * _BPW + j, 16:32]
        acc_v[j, 0:16] = a0
        acc_v[j, 16:32] = a1
    pltpu.sync_copy(acc_v, out_hbm.at[pl.ds(wid * _BPW, _BPW)])


def _sc_embed_sum(idx2, u):
    return pl.kernel(
        _sc_body,
        out_type=jax.ShapeDtypeStruct((BATCH_N, EMB_N), jnp.float32),
        mesh=plsc.VectorSubcoreMesh(core_axis_name="c", subcore_axis_name="s"),
        compiler_params=pltpu.CompilerParams(use_tc_tiling_on_sc=False),
        scratch_types=[
            pltpu.VMEM((CTX_N, _BPW), jnp.int32),
            pltpu.VMEM((_RPW, EMB_N), jnp.float32),
            pltpu.VMEM((_BPW, EMB_N), jnp.float32),
            pltpu.SemaphoreType.DMA,
        ],
    )(idx2, u)


def _lse_body(sums_ref, wt_ref, b_ref, lse_ref, s_ref):
    i = pl.program_id(0)

    @pl.when(i == 0)
    def _():
        s_ref[...] = jnp.zeros((1, BATCH_N), jnp.float32)

    @pl.when(i < _NT - 1)
    def _():
        x = lax.dot_general(
            wt_ref[...], sums_ref[...], (((0,), (1,)), ((), ())),
            preferred_element_type=jnp.float32,
        ) + b_ref[...].reshape(_VT, 1)
        s_ref[...] += jnp.sum(jnp.exp(x), axis=0, keepdims=True)

    @pl.when(i == _NT - 1)
    def _():
        v0 = i * _VT
        valid_c = (lax.broadcasted_iota(jnp.int32, (1, _VT), 1) + v0) < VOCAB_N
        wt = jnp.where(valid_c, wt_ref[...], 0.0)
        b = jnp.where(valid_c, b_ref[...], 0.0).reshape(_VT, 1)
        x = lax.dot_general(
            wt, sums_ref[...], (((0,), (1,)), ((), ())),
            preferred_element_type=jnp.float32,
        ) + b
        s = s_ref[...] + jnp.sum(jnp.exp(x), axis=0, keepdims=True)
        lse_ref[...] = jnp.log(s - jnp.float32(_PAD_COLS))


def _out_body(sums_ref, wt_ref, b_ref, lse_ref, o_ref):
    x = lax.dot_general(
        wt_ref[...], sums_ref[...], (((0,), (1,)), ((), ())),
        preferred_element_type=jnp.float32,
    )
    o_ref[...] = x + b_ref[...].reshape(_VT, 1) - lse_ref[...]


def kernel(inputs, U, W, b):
    idx2 = inputs.astype(jnp.int32)
    sums = _sc_embed_sum(idx2, U)
    wt = W.T
    b2 = b.reshape(1, VOCAB_N)
    lse = pl.pallas_call(
        _lse_body,
        grid=(_NT,),
        in_specs=[
            pl.BlockSpec((BATCH_N, EMB_N), lambda i: (0, 0)),
            pl.BlockSpec((EMB_N, _VT), lambda i: (0, i)),
            pl.BlockSpec((1, _VT), lambda i: (0, i)),
        ],
        out_specs=pl.BlockSpec((1, BATCH_N), lambda i: (0, 0)),
        out_shape=jax.ShapeDtypeStruct((1, BATCH_N), jnp.float32),
        scratch_shapes=[
            pltpu.VMEM((1, BATCH_N), jnp.float32),
        ],
    )(sums, wt, b2)
    out_t = pl.pallas_call(
        _out_body,
        grid=(_NT,),
        in_specs=[
            pl.BlockSpec((BATCH_N, EMB_N), lambda i: (0, 0)),
            pl.BlockSpec((EMB_N, _VT), lambda i: (0, i)),
            pl.BlockSpec((1, _VT), lambda i: (0, i)),
            pl.BlockSpec((1, BATCH_N), lambda i: (0, 0)),
        ],
        out_specs=pl.BlockSpec((_VT, BATCH_N), lambda i: (i, 0)),
        out_shape=jax.ShapeDtypeStruct((VOCAB_N, BATCH_N), jnp.float32),
    )(sums, wt, b2, lse)
    return out_t.T


# speedup vs baseline: 1.1483x; 1.0192x over previous
import jax
import jax.numpy as jnp
from jax import lax
from jax.experimental import pallas as pl
from jax.experimental.pallas import tpu as pltpu
from jax.experimental.pallas import tpu_sc as plsc

VOCAB_N = 100000
EMB_N = 32
CTX_N = 20
BATCH_N = 1024

_NC = 2
_NS = 16
_NW = _NC * _NS
_BPW = BATCH_N // _NW
_RPW = _BPW * CTX_N
_GCH = 128
_NG = _RPW // _GCH

_VT = 4096
_NT = 25
_PAD_COLS = _VT * _NT - VOCAB_N
_BT = 32
_NB = BATCH_N // _BT


def _sc_body(idx_hbm, u_hbm, out_hbm, idx_v, rows_v, acc_v, sem):
    wid = lax.axis_index("s") * _NC + lax.axis_index("c")
    pltpu.sync_copy(idx_hbm.at[:, pl.ds(wid * _BPW, _BPW)], idx_v)
    cps = [
        pltpu.async_copy(
            u_hbm.at[idx_v.at[c]], rows_v.at[pl.ds(c * _BPW, _BPW)], sem
        )
        for c in range(CTX_N)
    ]
    for cp in cps:
        cp.wait()
    for j in range(_BPW):
        a0 = rows_v[j, 0:16]
        a1 = rows_v[j, 16:32]
        for c in range(1, CTX_N):
            a0 = a0 + rows_v[c * _BPW + j, 0:16]
            a1 = a1 + rows_v[c * _BPW + j, 16:32]
        acc_v[j, 0:16] = a0
        acc_v[j, 16:32] = a1
    pltpu.sync_copy(acc_v, out_hbm.at[pl.ds(wid * _BPW, _BPW)])


def _sc_embed_sum(idx2, u):
    return pl.kernel(
        _sc_body,
        out_type=jax.ShapeDtypeStruct((BATCH_N, EMB_N), jnp.float32),
        mesh=plsc.VectorSubcoreMesh(core_axis_name="c", subcore_axis_name="s"),
        compiler_params=pltpu.CompilerParams(use_tc_tiling_on_sc=False),
        scratch_types=[
            pltpu.VMEM((CTX_N, _BPW), jnp.int32),
            pltpu.VMEM((_RPW, EMB_N), jnp.float32),
            pltpu.VMEM((_BPW, EMB_N), jnp.float32),
            pltpu.SemaphoreType.DMA,
        ],
    )(idx2, u)


def _lse_body(sums_ref, wt_ref, b_ref, lse_ref, s_ref):
    i = pl.program_id(0)

    @pl.when(i == 0)
    def _():
        s_ref[...] = jnp.zeros((1, BATCH_N), jnp.float32)

    @pl.when(i < _NT - 1)
    def _():
        x = lax.dot_general(
            wt_ref[...], sums_ref[...], (((0,), (1,)), ((), ())),
            preferred_element_type=jnp.float32,
        ) + b_ref[...].reshape(_VT, 1)
        s_ref[...] += jnp.sum(jnp.exp(x), axis=0, keepdims=True)

    @pl.when(i == _NT - 1)
    def _():
        v0 = i * _VT
        valid_c = (lax.broadcasted_iota(jnp.int32, (1, _VT), 1) + v0) < VOCAB_N
        wt = jnp.where(valid_c, wt_ref[...], 0.0)
        b = jnp.where(valid_c, b_ref[...], 0.0).reshape(_VT, 1)
        x = lax.dot_general(
            wt, sums_ref[...], (((0,), (1,)), ((), ())),
            preferred_element_type=jnp.float32,
        ) + b
        s = s_ref[...] + jnp.sum(jnp.exp(x), axis=0, keepdims=True)
        lse_ref[...] = jnp.log(s - jnp.float32(_PAD_COLS))


def _out_body(sums_ref, wt_ref, b_ref, lse_ref, o_ref):
    x = lax.dot_general(
        wt_ref[...], sums_ref[...], (((0,), (1,)), ((), ())),
        preferred_element_type=jnp.float32,
    )
    o_ref[...] = x + b_ref[...].reshape(_VT, 1) - lse_ref[...]


def kernel(inputs, U, W, b):
    idx2 = inputs.astype(jnp.int32)
    sums = _sc_embed_sum(idx2, U)
    wt = W.T
    b2 = b.reshape(1, VOCAB_N)
    lse = pl.pallas_call(
        _lse_body,
        grid=(_NT,),
        in_specs=[
            pl.BlockSpec((BATCH_N, EMB_N), lambda i: (0, 0)),
            pl.BlockSpec((EMB_N, _VT), lambda i: (0, i)),
            pl.BlockSpec((1, _VT), lambda i: (0, i)),
        ],
        out_specs=pl.BlockSpec((1, BATCH_N), lambda i: (0, 0)),
        out_shape=jax.ShapeDtypeStruct((1, BATCH_N), jnp.float32),
        scratch_shapes=[
            pltpu.VMEM((1, BATCH_N), jnp.float32),
        ],
    )(sums, wt, b2)
    out_t = pl.pallas_call(
        _out_body,
        grid=(_NT,),
        in_specs=[
            pl.BlockSpec((BATCH_N, EMB_N), lambda i: (0, 0)),
            pl.BlockSpec((EMB_N, _VT), lambda i: (0, i)),
            pl.BlockSpec((1, _VT), lambda i: (0, i)),
            pl.BlockSpec((1, BATCH_N), lambda i: (0, 0)),
        ],
        out_specs=pl.BlockSpec((_VT, BATCH_N), lambda i: (i, 0)),
        out_shape=jax.ShapeDtypeStruct((VOCAB_N, BATCH_N), jnp.float32),
    )(sums, wt, b2, lse)
    return out_t.T


# speedup vs baseline: 1.1520x; 1.0032x over previous
import jax
import jax.numpy as jnp
from jax import lax
from jax.experimental import pallas as pl
from jax.experimental.pallas import tpu as pltpu
from jax.experimental.pallas import tpu_sc as plsc

VOCAB_N = 100000
EMB_N = 32
CTX_N = 20
BATCH_N = 1024

_NC = 2
_NS = 16
_NW = _NC * _NS
_BPW = BATCH_N // _NW
_RPW = _BPW * CTX_N
_GCH = 128
_NG = _RPW // _GCH

_VT = 5120
_NT = 20
_PAD_COLS = _VT * _NT - VOCAB_N
_BT = 32
_NB = BATCH_N // _BT


def _sc_body(idx_hbm, u_hbm, out_hbm, idx_v, rows_v, acc_v, sem):
    wid = lax.axis_index("s") * _NC + lax.axis_index("c")
    pltpu.sync_copy(idx_hbm.at[:, pl.ds(wid * _BPW, _BPW)], idx_v)
    cps = [
        pltpu.async_copy(
            u_hbm.at[idx_v.at[c]], rows_v.at[pl.ds(c * _BPW, _BPW)], sem
        )
        for c in range(CTX_N)
    ]
    for cp in cps:
        cp.wait()
    for j in range(_BPW):
        a0 = rows_v[j, 0:16]
        a1 = rows_v[j, 16:32]
        for c in range(1, CTX_N):
            a0 = a0 + rows_v[c * _BPW + j, 0:16]
            a1 = a1 + rows_v[c * _BPW + j, 16:32]
        acc_v[j, 0:16] = a0
        acc_v[j, 16:32] = a1
    pltpu.sync_copy(acc_v, out_hbm.at[pl.ds(wid * _BPW, _BPW)])


def _sc_embed_sum(idx2, u):
    return pl.kernel(
        _sc_body,
        out_type=jax.ShapeDtypeStruct((BATCH_N, EMB_N), jnp.float32),
        mesh=plsc.VectorSubcoreMesh(core_axis_name="c", subcore_axis_name="s"),
        compiler_params=pltpu.CompilerParams(use_tc_tiling_on_sc=False),
        scratch_types=[
            pltpu.VMEM((CTX_N, _BPW), jnp.int32),
            pltpu.VMEM((_RPW, EMB_N), jnp.float32),
            pltpu.VMEM((_BPW, EMB_N), jnp.float32),
            pltpu.SemaphoreType.DMA,
        ],
    )(idx2, u)


def _lse_body(sums_ref, wt_ref, b_ref, lse_ref, s_ref):
    i = pl.program_id(0)

    @pl.when(i == 0)
    def _():
        s_ref[...] = jnp.zeros((1, BATCH_N), jnp.float32)

    @pl.when(i < _NT - 1)
    def _():
        x = lax.dot_general(
            wt_ref[...], sums_ref[...], (((0,), (1,)), ((), ())),
            preferred_element_type=jnp.float32,
        ) + b_ref[...].reshape(_VT, 1)
        s_ref[...] += jnp.sum(jnp.exp(x), axis=0, keepdims=True)

    @pl.when(i == _NT - 1)
    def _():
        v0 = i * _VT
        valid_c = (lax.broadcasted_iota(jnp.int32, (1, _VT), 1) + v0) < VOCAB_N
        wt = jnp.where(valid_c, wt_ref[...], 0.0)
        b = jnp.where(valid_c, b_ref[...], 0.0).reshape(_VT, 1)
        x = lax.dot_general(
            wt, sums_ref[...], (((0,), (1,)), ((), ())),
            preferred_element_type=jnp.float32,
        ) + b
        s = s_ref[...] + jnp.sum(jnp.exp(x), axis=0, keepdims=True)
        lse_ref[...] = jnp.log(s - jnp.float32(_PAD_COLS))


def _out_body(sums_ref, wt_ref, b_ref, lse_ref, o_ref):
    x = lax.dot_general(
        wt_ref[...], sums_ref[...], (((0,), (1,)), ((), ())),
        preferred_element_type=jnp.float32,
    )
    o_ref[...] = x + b_ref[...].reshape(_VT, 1) - lse_ref[...]


def kernel(inputs, U, W, b):
    idx2 = inputs.astype(jnp.int32)
    sums = _sc_embed_sum(idx2, U)
    wt = W.T
    b2 = b.reshape(1, VOCAB_N)
    lse = pl.pallas_call(
        _lse_body,
        grid=(_NT,),
        in_specs=[
            pl.BlockSpec((BATCH_N, EMB_N), lambda i: (0, 0)),
            pl.BlockSpec((EMB_N, _VT), lambda i: (0, i)),
            pl.BlockSpec((1, _VT), lambda i: (0, i)),
        ],
        out_specs=pl.BlockSpec((1, BATCH_N), lambda i: (0, 0)),
        out_shape=jax.ShapeDtypeStruct((1, BATCH_N), jnp.float32),
        scratch_shapes=[
            pltpu.VMEM((1, BATCH_N), jnp.float32),
        ],
    )(sums, wt, b2)
    out_t = pl.pallas_call(
        _out_body,
        grid=(_NT,),
        in_specs=[
            pl.BlockSpec((BATCH_N, EMB_N), lambda i: (0, 0)),
            pl.BlockSpec((EMB_N, _VT), lambda i: (0, i)),
            pl.BlockSpec((1, _VT), lambda i: (0, i)),
            pl.BlockSpec((1, BATCH_N), lambda i: (0, 0)),
        ],
        out_specs=pl.BlockSpec((_VT, BATCH_N), lambda i: (i, 0)),
        out_shape=jax.ShapeDtypeStruct((VOCAB_N, BATCH_N), jnp.float32),
    )(sums, wt, b2, lse)
    return out_t.T
